# baseline (speedup 1.0000x reference)
import functools

import numpy as np
import jax
import jax.numpy as jnp
from jax import lax
from jax.experimental import pallas as pl
from jax.experimental.pallas import tpu as pltpu
from jax.experimental.pallas import tpu_sc as plsc

_VOCAB = 1_000_000
_DIM = 32
_B = 16384
_L = 50
_LP = 128
_LG = 56
_N = _B * _L
_NC, _NS = 2, 16
_NW = _NC * _NS
_SEQ_PW = _B // _NW
_CSEQ = 16
_NCHUNK = _SEQ_PW // _CSEQ

_mask_words_np = None


def _threefry2x32(k0, k1, x0, x1):
    rot = [13, 15, 26, 6, 17, 29, 16, 24]
    ks = [np.uint32(k0), np.uint32(k1),
          np.uint32(np.uint32(k0) ^ np.uint32(k1) ^ np.uint32(0x1BD11BDA))]
    rotl = lambda v, r: (v << np.uint32(r)) | (v >> np.uint32(32 - r))
    x0 = x0 + ks[0]
    x1 = x1 + ks[1]
    for i in range(5):
        for r in (rot[0:4] if i % 2 == 0 else rot[4:8]):
            x0 = x0 + x1
            x1 = rotl(x1, r)
            x1 = x1 ^ x0
        x0 = x0 + ks[(i + 1) % 3]
        x1 = x1 + ks[(i + 2) % 3] + np.uint32(i + 1)
    return x0, x1


def _mask_words():
    global _mask_words_np
    if _mask_words_np is None:
        n = _N * _DIM
        with np.errstate(over="ignore"):
            o0, o1 = _threefry2x32(
                0, 42, np.zeros(n, dtype=np.uint32),
                np.arange(n, dtype=np.uint32))
        bits = ((o0 ^ o1) >> np.uint32(31)) == 0
        w = (bits.reshape(_N, _DIM).astype(np.uint32)
             << np.arange(_DIM, dtype=np.uint32)[None, :]).sum(
                 axis=1, dtype=np.uint32)
        wp = np.zeros((_B, _LP), dtype=np.uint32)
        wp[:, :_L] = w.reshape(_B, _L)
        _mask_words_np = wp.view(np.int32)
    return _mask_words_np


def _sc_embed_dropout(x2d, words, table):
    mesh = plsc.VectorSubcoreMesh(
        core_axis_name="c", subcore_axis_name="s",
        num_cores=_NC, num_subcores=_NS)

    @functools.partial(
        pl.kernel,
        out_type=jax.ShapeDtypeStruct((_B, _L, _DIM), jnp.float32),
        mesh=mesh,
        scratch_types=[
            pltpu.VMEM((_CSEQ, _LP), jnp.int32),
            pltpu.VMEM((_CSEQ, _LP), jnp.int32),
            pltpu.VMEM((_CSEQ, _LG, _DIM), jnp.float32),
            pltpu.VMEM((_CSEQ, _L, _DIM), jnp.float32),
            pltpu.SemaphoreType.DMA,
        ],
        compiler_params=pltpu.CompilerParams(use_tc_tiling_on_sc=False),
    )
    def body(x_hbm, words_hbm, table_hbm, out_hbm,
             idx_v, words_v, rows_v, out_v, sem):
        wid = lax.axis_index("s") * _NC + lax.axis_index("c")
        base = wid * _SEQ_PW
        iota0 = lax.iota(jnp.int32, 16)
        iota1 = iota0 + 16

        def chunk_body(g, carry):
            seq0 = base + g * _CSEQ
            pltpu.sync_copy(x_hbm.at[pl.ds(seq0, _CSEQ)], idx_v)
            pltpu.sync_copy(words_hbm.at[pl.ds(seq0, _CSEQ)], words_v)
            copies = [
                pltpu.async_copy(
                    table_hbm.at[idx_v.at[s, pl.ds(0, _LG)]],
                    rows_v.at[s], sem)
                for s in range(_CSEQ)
            ]
            for c in copies:
                c.wait()

            def seq_body(s, carry2):
                wv = [words_v[s, pl.ds(k * 16, 16)] for k in range(4)]
                for l in range(_L):
                    w = jnp.full((16,), wv[l // 16][l % 16],
                                 dtype=jnp.int32)
                    b0 = lax.shift_right_logical(w, iota0) & 1
                    b1 = lax.shift_right_logical(w, iota1) & 1
                    r0 = rows_v[s, l, pl.ds(0, 16)]
                    r1 = rows_v[s, l, pl.ds(16, 16)]
                    out_v[s, l, pl.ds(0, 16)] = jnp.where(
                        b0 != 0, r0 + r0, 0.0)
                    out_v[s, l, pl.ds(16, 16)] = jnp.where(
                        b1 != 0, r1 + r1, 0.0)
                return carry2

            lax.fori_loop(0, _CSEQ, seq_body, 0)
            pltpu.sync_copy(out_v, out_hbm.at[pl.ds(seq0, _CSEQ)])
            return carry

        lax.fori_loop(0, _NCHUNK, chunk_body, 0)

    return body(x2d, words, table)


def kernel(x, table):
    words = jnp.asarray(_mask_words())
    x_pad = jnp.pad(x, ((0, 0), (0, _LP - _L)), mode="edge")
    return _sc_embed_dropout(x_pad, words, table)


# speedup vs baseline: 1.1521x; 1.1521x over previous
import functools

import numpy as np
import jax
import jax.numpy as jnp
from jax import lax
from jax.experimental import pallas as pl
from jax.experimental.pallas import tpu as pltpu
from jax.experimental.pallas import tpu_sc as plsc

_VOCAB = 1_000_000
_DIM = 32
_B = 16384
_L = 50
_LP = 64
_N = _B * _L
_NC, _NS = 2, 16
_NW = _NC * _NS
_SEQ_PW = _B // _NW
_CSEQ = 16
_NCHUNK = _SEQ_PW // _CSEQ

_mask_words_np = None


def _threefry2x32(k0, k1, x0, x1):
    rot = [13, 15, 26, 6, 17, 29, 16, 24]
    ks = [np.uint32(k0), np.uint32(k1),
          np.uint32(np.uint32(k0) ^ np.uint32(k1) ^ np.uint32(0x1BD11BDA))]
    rotl = lambda v, r: (v << np.uint32(r)) | (v >> np.uint32(32 - r))
    x0 = x0 + ks[0]
    x1 = x1 + ks[1]
    for i in range(5):
        for r in (rot[0:4] if i % 2 == 0 else rot[4:8]):
            x0 = x0 + x1
            x1 = rotl(x1, r)
            x1 = x1 ^ x0
        x0 = x0 + ks[(i + 1) % 3]
        x1 = x1 + ks[(i + 2) % 3] + np.uint32(i + 1)
    return x0, x1


def _mask_words():
    global _mask_words_np
    if _mask_words_np is None:
        n = _N * _DIM
        with np.errstate(over="ignore"):
            o0, o1 = _threefry2x32(
                0, 42, np.zeros(n, dtype=np.uint32),
                np.arange(n, dtype=np.uint32))
        bits = ((o0 ^ o1) >> np.uint32(31)) == 0
        w = (bits.reshape(_N, _DIM).astype(np.uint32)
             << np.arange(_DIM, dtype=np.uint32)[None, :]).sum(
                 axis=1, dtype=np.uint32)
        wp = np.zeros((_B, _LP), dtype=np.uint32)
        wp[:, :_L] = w.reshape(_B, _L)
        _mask_words_np = wp.view(np.int32)
    return _mask_words_np


def _sc_embed_dropout(x2d, words, table):
    mesh = plsc.VectorSubcoreMesh(
        core_axis_name="c", subcore_axis_name="s",
        num_cores=_NC, num_subcores=_NS)

    @functools.partial(
        pl.kernel,
        out_type=jax.ShapeDtypeStruct((_B, _L, _DIM), jnp.float32),
        mesh=mesh,
        scratch_types=[
            pltpu.VMEM((_CSEQ, _L), jnp.int32),
            pltpu.VMEM((_CSEQ, _LP), jnp.int32),
            pltpu.VMEM((_CSEQ, _L, _DIM), jnp.float32),
            pltpu.SemaphoreType.DMA,
        ],
        compiler_params=pltpu.CompilerParams(use_tc_tiling_on_sc=False),
    )
    def body(x_hbm, words_hbm, table_hbm, out_hbm,
             idx_v, words_v, rows_v, sem):
        wid = lax.axis_index("s") * _NC + lax.axis_index("c")
        base = wid * _SEQ_PW
        iota0 = lax.iota(jnp.int32, 16)
        iota1 = iota0 + 16

        def chunk_body(g, carry):
            seq0 = base + g * _CSEQ
            pltpu.sync_copy(x_hbm.at[pl.ds(seq0, _CSEQ)], idx_v)
            pltpu.sync_copy(words_hbm.at[pl.ds(seq0, _CSEQ)], words_v)
            copies = [
                pltpu.async_copy(
                    table_hbm.at[idx_v.at[s]], rows_v.at[s], sem)
                for s in range(_CSEQ)
            ]
            for c in copies:
                c.wait()

            def seq_body(s, carry2):
                wv = [words_v[s, pl.ds(k * 16, 16)] for k in range(4)]
                for l in range(_L):
                    w = jnp.full((16,), wv[l // 16][l % 16],
                                 dtype=jnp.int32)
                    b0 = lax.shift_right_logical(w, iota0) & 1
                    b1 = lax.shift_right_logical(w, iota1) & 1
                    r0 = rows_v[s, l, pl.ds(0, 16)]
                    r1 = rows_v[s, l, pl.ds(16, 16)]
                    rows_v[s, l, pl.ds(0, 16)] = jnp.where(
                        b0 != 0, r0 + r0, 0.0)
                    rows_v[s, l, pl.ds(16, 16)] = jnp.where(
                        b1 != 0, r1 + r1, 0.0)
                return carry2

            lax.fori_loop(0, _CSEQ, seq_body, 0)
            pltpu.sync_copy(rows_v, out_hbm.at[pl.ds(seq0, _CSEQ)])
            return carry

        lax.fori_loop(0, _NCHUNK, chunk_body, 0)

    return body(x2d, words, table)


_words_dev = None


def kernel(x, table):
    global _words_dev
    if _words_dev is None:
        _words_dev = jax.device_put(_mask_words())
    return _sc_embed_dropout(x, _words_dev, table)


# speedup vs baseline: 1.1533x; 1.0010x over previous
import functools

import numpy as np
import jax
import jax.numpy as jnp
from jax import lax
from jax.experimental import pallas as pl
from jax.experimental.pallas import tpu as pltpu
from jax.experimental.pallas import tpu_sc as plsc
from jax.experimental import layout

_VOCAB = 1_000_000
_DIM = 32
_B = 16384
_L = 50
_LP = 64
_N = _B * _L
_NC, _NS = 2, 16
_NW = _NC * _NS
_SEQ_PW = _B // _NW
_CSEQ = 16
_NCHUNK = _SEQ_PW // _CSEQ

_mask_words_np = None


def _threefry2x32(k0, k1, x0, x1):
    rot = [13, 15, 26, 6, 17, 29, 16, 24]
    ks = [np.uint32(k0), np.uint32(k1),
          np.uint32(np.uint32(k0) ^ np.uint32(k1) ^ np.uint32(0x1BD11BDA))]
    rotl = lambda v, r: (v << np.uint32(r)) | (v >> np.uint32(32 - r))
    x0 = x0 + ks[0]
    x1 = x1 + ks[1]
    for i in range(5):
        for r in (rot[0:4] if i % 2 == 0 else rot[4:8]):
            x0 = x0 + x1
            x1 = rotl(x1, r)
            x1 = x1 ^ x0
        x0 = x0 + ks[(i + 1) % 3]
        x1 = x1 + ks[(i + 2) % 3] + np.uint32(i + 1)
    return x0, x1


def _mask_words():
    global _mask_words_np
    if _mask_words_np is None:
        n = _N * _DIM
        with np.errstate(over="ignore"):
            o0, o1 = _threefry2x32(
                0, 42, np.zeros(n, dtype=np.uint32),
                np.arange(n, dtype=np.uint32))
        bits = ((o0 ^ o1) >> np.uint32(31)) == 0
        w = (bits.reshape(_N, _DIM).astype(np.uint32)
             << np.arange(_DIM, dtype=np.uint32)[None, :]).sum(
                 axis=1, dtype=np.uint32)
        wp = np.zeros((_B, _LP), dtype=np.uint32)
        wp[:, :_L] = w.reshape(_B, _L)
        _mask_words_np = wp.view(np.int32)
    return _mask_words_np


def _sc_embed_dropout(x2d, words, table):
    mesh = plsc.VectorSubcoreMesh(
        core_axis_name="c", subcore_axis_name="s",
        num_cores=_NC, num_subcores=_NS)

    @functools.partial(
        pl.kernel,
        out_type=jax.ShapeDtypeStruct((_B, _L, _DIM), jnp.float32),
        mesh=mesh,
        scratch_types=[
            pltpu.VMEM((_CSEQ, _L), jnp.int32),
            pltpu.VMEM((_CSEQ, _LP), jnp.int32),
            pltpu.VMEM((_CSEQ, _L, _DIM), jnp.float32),
            pltpu.SemaphoreType.DMA,
        ],
        compiler_params=pltpu.CompilerParams(use_tc_tiling_on_sc=False),
    )
    def body(x_hbm, words_hbm, table_hbm, out_hbm,
             idx_v, words_v, rows_v, sem):
        wid = lax.axis_index("s") * _NC + lax.axis_index("c")
        base = wid * _SEQ_PW
        iota0 = lax.iota(jnp.int32, 16)
        iota1 = iota0 + 16

        def chunk_body(g, carry):
            seq0 = base + g * _CSEQ
            pltpu.sync_copy(x_hbm.at[pl.ds(seq0, _CSEQ)], idx_v)
            pltpu.sync_copy(words_hbm.at[pl.ds(seq0, _CSEQ)], words_v)
            copies = [
                pltpu.async_copy(
                    table_hbm.at[idx_v.at[s]], rows_v.at[s], sem)
                for s in range(_CSEQ)
            ]
            for c in copies:
                c.wait()

            def seq_body(s, carry2):
                wv = [words_v[s, pl.ds(k * 16, 16)] for k in range(4)]
                for l in range(_L):
                    w = jnp.full((16,), wv[l // 16][l % 16],
                                 dtype=jnp.int32)
                    b0 = lax.shift_right_logical(w, iota0) & 1
                    b1 = lax.shift_right_logical(w, iota1) & 1
                    r0 = rows_v[s, l, pl.ds(0, 16)]
                    r1 = rows_v[s, l, pl.ds(16, 16)]
                    rows_v[s, l, pl.ds(0, 16)] = jnp.where(
                        b0 != 0, r0 + r0, 0.0)
                    rows_v[s, l, pl.ds(16, 16)] = jnp.where(
                        b1 != 0, r1 + r1, 0.0)
                return carry2

            lax.fori_loop(0, _CSEQ, seq_body, 0)
            pltpu.sync_copy(rows_v, out_hbm.at[pl.ds(seq0, _CSEQ)])
            return carry

        lax.fori_loop(0, _NCHUNK, chunk_body, 0)

    return body(x2d, words, table)


_words_dev = None


def kernel(x, table):
    global _words_dev
    if _words_dev is None:
        _words_dev = jax.device_put(_mask_words())
    out = _sc_embed_dropout(x, _words_dev, table)
    out = layout.with_layout_constraint(
        out, layout.Layout((0, 1, 2), ((8,),)))
    return layout.with_layout_constraint(
        out, layout.Layout((1, 2, 0), ((8, 128),)))


# speedup vs baseline: 1.2121x; 1.0510x over previous
import functools

import numpy as np
import jax
import jax.numpy as jnp
from jax import lax
from jax.experimental import pallas as pl
from jax.experimental.pallas import tpu as pltpu
from jax.experimental.pallas import tpu_sc as plsc

_VOCAB = 1_000_000
_DIM = 32
_B = 16384
_L = 50
_LP = 64
_N = _B * _L
_NC, _NS = 2, 16
_NW = _NC * _NS
_SEQ_PW = _B // _NW
_CSEQ = 16
_NCHUNK = _SEQ_PW // _CSEQ

_mask_words_np = None


def _threefry2x32(k0, k1, x0, x1):
    rot = [13, 15, 26, 6, 17, 29, 16, 24]
    ks = [np.uint32(k0), np.uint32(k1),
          np.uint32(np.uint32(k0) ^ np.uint32(k1) ^ np.uint32(0x1BD11BDA))]
    rotl = lambda v, r: (v << np.uint32(r)) | (v >> np.uint32(32 - r))
    x0 = x0 + ks[0]
    x1 = x1 + ks[1]
    for i in range(5):
        for r in (rot[0:4] if i % 2 == 0 else rot[4:8]):
            x0 = x0 + x1
            x1 = rotl(x1, r)
            x1 = x1 ^ x0
        x0 = x0 + ks[(i + 1) % 3]
        x1 = x1 + ks[(i + 2) % 3] + np.uint32(i + 1)
    return x0, x1


def _mask_words():
    global _mask_words_np
    if _mask_words_np is None:
        n = _N * _DIM
        with np.errstate(over="ignore"):
            o0, o1 = _threefry2x32(
                0, 42, np.zeros(n, dtype=np.uint32),
                np.arange(n, dtype=np.uint32))
        bits = ((o0 ^ o1) >> np.uint32(31)) == 0
        w = (bits.reshape(_N, _DIM).astype(np.uint32)
             << np.arange(_DIM, dtype=np.uint32)[None, :]).sum(
                 axis=1, dtype=np.uint32)
        wp = np.zeros((_B, _LP), dtype=np.uint32)
        wp[:, :_L] = w.reshape(_B, _L)
        _mask_words_np = wp.view(np.int32)
    return _mask_words_np


def _sc_embed_dropout(x2d, words, table):
    mesh = plsc.VectorSubcoreMesh(
        core_axis_name="c", subcore_axis_name="s",
        num_cores=_NC, num_subcores=_NS)

    @functools.partial(
        pl.kernel,
        out_type=jax.ShapeDtypeStruct((_B, _L, _DIM), jnp.float32),
        mesh=mesh,
        scratch_types=[
            pltpu.VMEM((_CSEQ, _L), jnp.int32),
            pltpu.VMEM((_CSEQ, _LP), jnp.int32),
            pltpu.VMEM((_CSEQ, _L, _DIM), jnp.float32),
            pltpu.VMEM((_CSEQ, _L), jnp.int32),
            pltpu.VMEM((_CSEQ, _LP), jnp.int32),
            pltpu.VMEM((_CSEQ, _L, _DIM), jnp.float32),
            pltpu.SemaphoreType.DMA,
            pltpu.SemaphoreType.DMA,
        ],
        compiler_params=pltpu.CompilerParams(use_tc_tiling_on_sc=False),
    )
    def body(x_hbm, words_hbm, table_hbm, out_hbm,
             idx_a, words_a, rows_a, idx_b, words_b, rows_b, sem_a, sem_b):
        wid = lax.axis_index("s") * _NC + lax.axis_index("c")
        base = wid * _SEQ_PW
        iota0 = lax.iota(jnp.int32, 16)
        iota1 = iota0 + 16

        def load_and_fire(g, idx_v, words_v, rows_v, sem):
            seq0 = base + g * _CSEQ
            pltpu.sync_copy(x_hbm.at[pl.ds(seq0, _CSEQ)], idx_v)
            pltpu.sync_copy(words_hbm.at[pl.ds(seq0, _CSEQ)], words_v)
            for s in range(_CSEQ):
                pltpu.async_copy(
                    table_hbm.at[idx_v.at[s]], rows_v.at[s], sem)

        def wait_rows(g, rows_v, sem):
            seq0 = base + g * _CSEQ
            pltpu.make_async_copy(
                out_hbm.at[pl.ds(seq0, _CSEQ)], rows_v, sem).wait()

        def compute_and_store(g, words_v, rows_v):
            def seq_body(s, carry2):
                wv = [words_v[s, pl.ds(k * 16, 16)] for k in range(4)]
                for l in range(_L):
                    w = jnp.full((16,), wv[l // 16][l % 16],
                                 dtype=jnp.int32)
                    b0 = lax.shift_right_logical(w, iota0) & 1
                    b1 = lax.shift_right_logical(w, iota1) & 1
                    r0 = rows_v[s, l, pl.ds(0, 16)]
                    r1 = rows_v[s, l, pl.ds(16, 16)]
                    rows_v[s, l, pl.ds(0, 16)] = jnp.where(
                        b0 != 0, r0 + r0, 0.0)
                    rows_v[s, l, pl.ds(16, 16)] = jnp.where(
                        b1 != 0, r1 + r1, 0.0)
                return carry2

            lax.fori_loop(0, _CSEQ, seq_body, 0)
            seq0 = base + g * _CSEQ
            pltpu.sync_copy(rows_v, out_hbm.at[pl.ds(seq0, _CSEQ)])

        load_and_fire(0, idx_a, words_a, rows_a, sem_a)

        def pair_body(g2, carry):
            ca = 2 * g2
            cb = 2 * g2 + 1
            load_and_fire(cb, idx_b, words_b, rows_b, sem_b)
            wait_rows(ca, rows_a, sem_a)
            compute_and_store(ca, words_a, rows_a)

            @pl.when(g2 + 1 < _NCHUNK // 2)
            def _():
                load_and_fire(ca + 2, idx_a, words_a, rows_a, sem_a)

            wait_rows(cb, rows_b, sem_b)
            compute_and_store(cb, words_b, rows_b)
            return carry

        lax.fori_loop(0, _NCHUNK // 2, pair_body, 0)

    return body(x2d, words, table)


_words_dev = None


def kernel(x, table):
    global _words_dev
    if _words_dev is None:
        _words_dev = jax.device_put(_mask_words())
    return _sc_embed_dropout(x, _words_dev, table)
